# baseline (speedup 1.0000x reference)
import jax
import jax.numpy as jnp
from jax.experimental import pallas as pl


def _rotation_consts(B, C, dtype):
    key = jax.random.key(42)
    _, k_flip, k_perm = jax.random.split(key, 3)
    flip_index = jax.random.bernoulli(k_flip, 0.5, (B * C,)).astype(jnp.int32)
    ones = jnp.ones(B * C, dtype=dtype)
    flip = jnp.where(flip_index == 0, -ones, ones).reshape(B, 1, C)
    rotate_axis = jax.random.permutation(k_perm, C).astype(jnp.int32)
    return flip, rotate_axis.reshape(1, C)


def _body(x_ref, s_ref, p_ref, o_ref):
    xb = x_ref[0]
    idx = jnp.broadcast_to(p_ref[0][None, :], xb.shape)
    g = jnp.take_along_axis(xb, idx, axis=1)
    o_ref[0] = g * s_ref[0]


def _tc_half(x, flip, perm):
    Bh, T, C = x.shape
    TT = 512
    grid = (Bh, T // TT)
    return pl.pallas_call(
        _body,
        grid=grid,
        in_specs=[
            pl.BlockSpec((1, TT, C), lambda b, t: (b, t, 0)),
            pl.BlockSpec((1, 1, C), lambda b, t: (b, 0, 0)),
            pl.BlockSpec((1, C), lambda b, t: (0, 0)),
        ],
        out_specs=pl.BlockSpec((1, TT, C), lambda b, t: (b, t, 0)),
        out_shape=jax.ShapeDtypeStruct((Bh, T, C), x.dtype),
    )(x, flip, perm)


def kernel(x):
    B, T, C = x.shape
    flip, perm = _rotation_consts(B, C, x.dtype)
    H = B // 2
    lo = _tc_half(x[:H], flip[:H], perm)
    hi = _tc_half(x[H:], flip[H:], perm)
    return jnp.concatenate([lo, hi], axis=0)


# speedup vs baseline: 1.5545x; 1.5545x over previous
import functools
import jax
import jax.numpy as jnp
from jax import lax
from jax.experimental import pallas as pl
from jax.experimental.pallas import tpu as pltpu
from jax.experimental.pallas import tpu_sc as plsc

_NC, _NS, _L = 2, 16, 16
_NW = _NC * _NS


def _rotation_consts(B, C, dtype):
    key = jax.random.key(42)
    _, k_flip, k_perm = jax.random.split(key, 3)
    flip_index = jax.random.bernoulli(k_flip, 0.5, (B * C,)).astype(jnp.int32)
    ones = jnp.ones(B * C, dtype=dtype)
    flip = jnp.where(flip_index == 0, -ones, ones)
    rotate_axis = jax.random.permutation(k_perm, C).astype(jnp.int32)
    return flip, rotate_axis


def kernel(x):
    B, T, C = x.shape
    flip, perm = _rotation_consts(B, C, x.dtype)
    G = C // _L
    R = 128
    NCH = T // R
    BPW = B // _NW
    TOT = BPW * NCH
    RC = R * C
    xf = x.reshape(B, T * C)
    mesh = plsc.VectorSubcoreMesh(
        core_axis_name="c", subcore_axis_name="s", num_cores=_NC, num_subcores=_NS
    )

    @functools.partial(
        pl.kernel,
        mesh=mesh,
        out_type=jax.ShapeDtypeStruct((B, T * C), x.dtype),
        scratch_types=[
            pltpu.VMEM((RC,), jnp.float32),
            pltpu.VMEM((RC,), jnp.float32),
            pltpu.VMEM((RC,), jnp.float32),
            pltpu.VMEM((RC,), jnp.float32),
            pltpu.VMEM((BPW * C,), jnp.float32),
            pltpu.VMEM((C,), jnp.int32),
            pltpu.SemaphoreType.DMA,
            pltpu.SemaphoreType.DMA,
            pltpu.SemaphoreType.DMA,
            pltpu.SemaphoreType.DMA,
        ],
        compiler_params=pltpu.CompilerParams(needs_layout_passes=False),
    )
    def run(x_hbm, s_hbm, p_hbm, o_hbm, in0, in1, out0, out1, s_v, p_v,
            si0, si1, so0, so1):
        inb = (in0, in1)
        outb = (out0, out1)
        sin = (si0, si1)
        sout = (so0, so1)
        wid = lax.axis_index("s") * _NC + lax.axis_index("c")
        base = wid * BPW
        pltpu.sync_copy(p_hbm, p_v)
        pltpu.sync_copy(s_hbm.at[pl.ds(base * C, BPW * C)], s_v)
        pgs = [p_v[pl.ds(g * _L, _L)] for g in range(G)]
        sgs = [[s_v[pl.ds(k * C + g * _L, _L)] for g in range(G)]
               for k in range(BPW)]

        def src(ch):
            k = ch // NCH
            return base + k, (ch - k * NCH) * RC

        b0, o0 = src(0)
        pltpu.async_copy(x_hbm.at[b0, pl.ds(o0, RC)], inb[0], sin[0])
        b1, o1 = src(1)
        pltpu.async_copy(x_hbm.at[b1, pl.ds(o1, RC)], inb[1], sin[1])

        def half(p, cur, ch):
            bc, oc = src(ch)
            pltpu.make_async_copy(
                x_hbm.at[bc, pl.ds(oc, RC)], inb[cur], sin[cur]
            ).wait()

            @pl.when(p > 0)
            def _():
                pltpu.make_async_copy(
                    outb[cur], o_hbm.at[bc, pl.ds(oc, RC)], sout[cur]
                ).wait()

            kb = ch // NCH
            sg = [jnp.where(kb == 0, sgs[0][g], sgs[1][g]) for g in range(G)]
            ib = inb[cur]
            ob = outb[cur]

            @plsc.parallel_loop(0, R, step=1, unroll=8, carry=tuple(pgs))
            def _rows(r, idxs):
                rbase = r * C
                for g in range(G):
                    v = plsc.load_gather(ib, [idxs[g]])
                    ob[pl.ds(rbase + g * _L, _L)] = v * sg[g]
                return tuple(ix + C for ix in idxs)

            pltpu.async_copy(outb[cur], o_hbm.at[bc, pl.ds(oc, RC)], sout[cur])

            @pl.when(ch + 2 < TOT)
            def _():
                bn, on = src(ch + 2)
                pltpu.async_copy(x_hbm.at[bn, pl.ds(on, RC)], inb[cur], sin[cur])

        def pair(p, carry):
            half(p, 0, 2 * p)
            half(p, 1, 2 * p + 1)
            return carry

        lax.fori_loop(0, TOT // 2, pair, 0)
        for ch in (TOT - 2, TOT - 1):
            cur = ch % 2
            bc, oc = src(ch)
            pltpu.make_async_copy(
                outb[cur], o_hbm.at[bc, pl.ds(oc, RC)], sout[cur]
            ).wait()

    return run(xf, flip.reshape(B * C), perm).reshape(B, T, C)


# speedup vs baseline: 3.2559x; 2.0944x over previous
import functools
import jax
import jax.numpy as jnp
from jax import lax
from jax.experimental import pallas as pl
from jax.experimental.pallas import tpu as pltpu
from jax.experimental.pallas import tpu_sc as plsc

_NC, _NS, _L = 2, 16, 16
_NW = _NC * _NS


def _rotation_consts(B, C, dtype):
    key = jax.random.key(42)
    _, k_flip, k_perm = jax.random.split(key, 3)
    flip_index = jax.random.bernoulli(k_flip, 0.5, (B * C,)).astype(jnp.int32)
    ones = jnp.ones(B * C, dtype=dtype)
    flip = jnp.where(flip_index == 0, -ones, ones)
    rotate_axis = jax.random.permutation(k_perm, C).astype(jnp.int32)
    return flip, rotate_axis


def kernel(x):
    B, T, C = x.shape
    flip, perm = _rotation_consts(B, C, x.dtype)
    G = C // _L
    R = 128
    NCH = T // R
    BPW = B // _NW
    TOT = BPW * NCH
    mesh = plsc.VectorSubcoreMesh(
        core_axis_name="c", subcore_axis_name="s", num_cores=_NC, num_subcores=_NS
    )

    @functools.partial(
        pl.kernel,
        mesh=mesh,
        out_type=jax.ShapeDtypeStruct((B, T, C), x.dtype),
        scratch_types=[
            pltpu.VMEM((R, C), jnp.float32),
            pltpu.VMEM((R, C), jnp.float32),
            pltpu.VMEM((R, C), jnp.float32),
            pltpu.VMEM((R, C), jnp.float32),
            pltpu.VMEM((BPW * C,), jnp.float32),
            pltpu.VMEM((C,), jnp.int32),
            pltpu.SemaphoreType.DMA,
            pltpu.SemaphoreType.DMA,
            pltpu.SemaphoreType.DMA,
            pltpu.SemaphoreType.DMA,
        ],
        compiler_params=pltpu.CompilerParams(needs_layout_passes=False),
    )
    def run(x_hbm, s_hbm, p_hbm, o_hbm, in0, in1, out0, out1, s_v, p_v,
            si0, si1, so0, so1):
        inb = (in0, in1)
        outb = (out0, out1)
        sin = (si0, si1)
        sout = (so0, so1)
        wid = lax.axis_index("s") * _NC + lax.axis_index("c")
        base = wid * BPW
        pltpu.sync_copy(p_hbm, p_v)
        pltpu.sync_copy(s_hbm.at[pl.ds(base * C, BPW * C)], s_v)
        pgs = [p_v[pl.ds(g * _L, _L)] for g in range(G)]
        sgs = [[s_v[pl.ds(k * C + g * _L, _L)] for g in range(G)]
               for k in range(BPW)]
        zrow = jax.lax.broadcast(jnp.int32(0), (_L,))
        one = jnp.int32(1)

        def src(ch):
            k = ch // NCH
            return base + k, (ch - k * NCH) * R

        b0, o0 = src(0)
        pltpu.async_copy(x_hbm.at[b0, pl.ds(o0, R)], inb[0], sin[0])
        b1, o1 = src(1)
        pltpu.async_copy(x_hbm.at[b1, pl.ds(o1, R)], inb[1], sin[1])

        def half(p, cur, ch):
            bc, oc = src(ch)
            pltpu.make_async_copy(
                x_hbm.at[bc, pl.ds(oc, R)], inb[cur], sin[cur]
            ).wait()

            @pl.when(p > 0)
            def _():
                pltpu.make_async_copy(
                    outb[cur], o_hbm.at[bc, pl.ds(oc, R)], sout[cur]
                ).wait()

            kb = ch // NCH
            sg = [jnp.where(kb == 0, sgs[0][g], sgs[1][g]) for g in range(G)]
            ib = inb[cur]
            ob = outb[cur]

            @plsc.parallel_loop(0, R, step=1, unroll=8, carry=zrow)
            def _rows(r, rs):
                for g in range(G):
                    v = plsc.load_gather(ib, [rs, pgs[g]])
                    ob[r, pl.ds(g * _L, _L)] = v * sg[g]
                return rs + one

            pltpu.async_copy(outb[cur], o_hbm.at[bc, pl.ds(oc, R)], sout[cur])

            @pl.when(ch + 2 < TOT)
            def _():
                bn, on = src(ch + 2)
                pltpu.async_copy(x_hbm.at[bn, pl.ds(on, R)], inb[cur], sin[cur])

        def pair(p, carry):
            half(p, 0, 2 * p)
            half(p, 1, 2 * p + 1)
            return carry

        lax.fori_loop(0, TOT // 2, pair, 0)
        for ch in (TOT - 2, TOT - 1):
            cur = ch % 2
            bc, oc = src(ch)
            pltpu.make_async_copy(
                outb[cur], o_hbm.at[bc, pl.ds(oc, R)], sout[cur]
            ).wait()

    return run(x, flip.reshape(B * C), perm)
